# baseline (speedup 1.0000x reference)
import functools

import jax
import jax.numpy as jnp
from jax import lax
from jax.experimental import pallas as pl
from jax.experimental.pallas import tpu as pltpu
from jax.experimental.pallas import tpu_sc as plsc

N_E = 8192
E_DIM = 256
BETA = 0.25
ENT_RATIO = 0.1
TEMP = 0.01

N_TOK = 4096
BM = 256
KC = 2048
N_KC = N_E // KC
GRID = N_TOK // BM
BLOCKS_PER_BATCH = 1024 // BM

_PREC = lax.Precision.DEFAULT


def _vq_main_body(qids_ref, zt_ref, w_ref,
                  zn_ref, idx_ref, vq_ref, com_ref, ent_ref, dcr_ref,
                  e_s, emb_s, ne2_s, colacc, ref0, eqacc, sacc):
    pb = pl.program_id(0)
    b = pl.program_id(1)

    @pl.when((pb == 0) & (b == 0))
    def _init():
        colacc[...] = jnp.zeros_like(colacc)
        sacc[0] = 0.0
        sacc[1] = 0.0
        sacc[2] = 0.0
        sacc[3] = 0.0
        for c in range(N_KC):
            wc = w_ref[pl.ds(c * KC, KC), :]
            nw = jnp.sqrt(jnp.sum(wc * wc, axis=1, keepdims=True))
            ec = wc / jnp.maximum(nw, 1e-12)
            emb_s[pl.ds(c * KC, KC), :] = ec
            ne2_s[0, pl.ds(c * KC, KC)] = jnp.sum(ec * ec, axis=1)

    zc = zt_ref[0]
    z = zc.T
    nz = jnp.sqrt(jnp.sum(z * z, axis=1, keepdims=True))
    zn = z / jnp.maximum(nz, 1e-12)
    zn_ref[...] = zn
    nz2 = jnp.sum(zn * zn, axis=1)

    zn2 = zn + zn
    iota_l = lax.broadcasted_iota(jnp.int32, (BM, KC), 1)
    ones_col = jnp.ones((KC, 1), jnp.float32)
    _C2 = -1.4426950408889634 / TEMP
    run_min = None
    run_arg = None
    rm_list = []
    Z = None
    Sed = None
    for kb in range(N_KC):
        emb_c = emb_s[pl.ds(kb * KC, KC), :]
        s2 = lax.dot_general(zn2, emb_c, (((1,), (1,)), ((), ())),
                             preferred_element_type=jnp.float32,
                             precision=_PREC)
        d_c = nz2[:, None] + ne2_s[0, pl.ds(kb * KC, KC)][None, :] - s2
        mn_c = jnp.min(d_c, axis=1)
        am_c = jnp.min(jnp.where(d_c == mn_c[:, None], iota_l, jnp.int32(2**30)),
                       axis=1) + kb * KC
        rm_prev = run_min
        if kb == 0:
            run_min, run_arg = mn_c, am_c
        else:
            upd = mn_c < run_min
            run_min = jnp.where(upd, mn_c, run_min)
            run_arg = jnp.where(upd, am_c, run_arg)
        rm_list.append(run_min)
        e_c = jnp.exp2((d_c - run_min[:, None]) * _C2)
        e_s[:, pl.ds(kb * KC, KC)] = e_c.astype(jnp.bfloat16)
        z_c = lax.dot_general(e_c, ones_col, (((1,), (0,)), ((), ())),
                              preferred_element_type=jnp.float32)[:, 0]
        sed_c = jnp.sum(e_c * d_c, axis=1)
        if kb == 0:
            Z, Sed = z_c, sed_c
        else:
            fac = jnp.exp((run_min - rm_prev) * (1.0 / TEMP))
            Z = Z * fac + z_c
            Sed = Sed * fac + sed_c

    rz = 1.0 / Z
    for kb in range(N_KC):
        w = (jnp.exp((run_min - rm_list[kb]) * (1.0 / TEMP)) * rz)
        w_bf = w.astype(jnp.bfloat16)[None, :]
        e_c = e_s[:, pl.ds(kb * KC, KC)]
        part = lax.dot_general(w_bf, e_c, (((1,), (0,)), ((), ())),
                               preferred_element_type=jnp.float32)
        colacc[0, pl.ds(kb * KC, KC)] = colacc[0, pl.ds(kb * KC, KC)] + part[0]

    plogp = (Sed / Z - run_min) * (-1.0 / TEMP) - jnp.log(Z)
    sacc[0] = sacc[0] + jnp.sum(plogp)

    w_coord = lax.broadcasted_iota(jnp.int32, (1, BM), 1)[0] % 32
    q = qids_ref[b]
    maskv = (w_coord <= q).astype(jnp.float32)
    sacc[1] = sacc[1] + jnp.sum(run_min * maskv)
    sacc[2] = sacc[2] + jnp.sum(maskv)

    idx_ref[0, 0, :] = run_arg

    @pl.when(b == 0)
    def _dcr_first():
        ref0[0, :] = run_arg
        eqacc[0, :] = jnp.ones((BM,), jnp.int32)

    @pl.when(b > 0)
    def _dcr_rest():
        eqacc[0, :] = eqacc[0, :] & (run_arg == ref0[0, :]).astype(jnp.int32)

    @pl.when(b == 3)
    def _dcr_sum():
        sacc[3] = sacc[3] + jnp.sum(eqacc[0, :].astype(jnp.float32))

    @pl.when((pb == BLOCKS_PER_BATCH - 1) & (b == 3))
    def _final():
        avgp = colacc[0, :] / jnp.float32(N_TOK)
        avg_entropy = -jnp.sum(avgp * jnp.log(avgp + 1e-5))
        sample_entropy = -(sacc[0] / jnp.float32(N_TOK))
        ent_ref[0, 0] = ENT_RATIO * (sample_entropy - avg_entropy)
        vq_loss = (sacc[1] / jnp.float32(E_DIM)) / (sacc[2] / 32.0)
        vq_ref[0, 0] = vq_loss
        com_ref[0, 0] = BETA * vq_loss
        dcr_ref[0, 0] = sacc[3] / jnp.float32(1024)


def _vq_main(z3, W, qids_flat):
    scalar_spec = pl.BlockSpec(memory_space=pltpu.SMEM)
    def _tok(pb, b):
        return b * BLOCKS_PER_BATCH + pb

    grid_spec = pltpu.PrefetchScalarGridSpec(
        num_scalar_prefetch=1,
        grid=(BLOCKS_PER_BATCH, 4),
        in_specs=[
            pl.BlockSpec((1, E_DIM, BM), lambda pb, b, q: (b, 0, pb)),
            pl.BlockSpec((N_E, E_DIM), lambda pb, b, q: (0, 0)),
        ],
        out_specs=[
            pl.BlockSpec((BM, E_DIM), lambda pb, b, q: (_tok(pb, b), 0)),
            pl.BlockSpec((1, 1, BM), lambda pb, b, q: (_tok(pb, b), 0, 0)),
            scalar_spec, scalar_spec, scalar_spec, scalar_spec,
        ],
        scratch_shapes=[
            pltpu.VMEM((BM, N_E), jnp.bfloat16),
            pltpu.VMEM((N_E, E_DIM), jnp.float32),
            pltpu.VMEM((1, N_E), jnp.float32),
            pltpu.VMEM((1, N_E), jnp.float32),
            pltpu.VMEM((1, BM), jnp.int32),
            pltpu.VMEM((1, BM), jnp.int32),
            pltpu.SMEM((4,), jnp.float32),
        ],
    )
    return pl.pallas_call(
        _vq_main_body,
        grid_spec=grid_spec,
        out_shape=[
            jax.ShapeDtypeStruct((N_TOK, E_DIM), jnp.float32),
            jax.ShapeDtypeStruct((GRID, 1, BM), jnp.int32),
            jax.ShapeDtypeStruct((1, 1), jnp.float32),
            jax.ShapeDtypeStruct((1, 1), jnp.float32),
            jax.ShapeDtypeStruct((1, 1), jnp.float32),
            jax.ShapeDtypeStruct((1, 1), jnp.float32),
        ],
    )(qids_flat, z3, W)


def _sc_gather(emb, idx):
    info = plsc.get_sparse_core_info()
    nc, ns = info.num_cores, info.num_subcores
    nw = nc * ns
    b_per_w = N_TOK // nw
    mesh = plsc.VectorSubcoreMesh(core_axis_name="c", subcore_axis_name="s")

    @functools.partial(
        pl.kernel, mesh=mesh,
        out_type=jax.ShapeDtypeStruct((N_TOK, E_DIM), jnp.float32),
        scratch_types=[
            pltpu.VMEM((b_per_w,), jnp.int32),
            pltpu.VMEM((b_per_w, E_DIM), jnp.float32),
            pltpu.SemaphoreType.DMA,
        ],
    )
    def k(table_hbm, idx_hbm, out_hbm, idx_v, rows_v, sem):
        wid = lax.axis_index("s") * nc + lax.axis_index("c")
        base = wid * b_per_w
        pltpu.sync_copy(idx_hbm.at[pl.ds(base, b_per_w)], idx_v)
        pltpu.async_copy(table_hbm.at[idx_v], rows_v, sem).wait()
        pltpu.sync_copy(rows_v, out_hbm.at[pl.ds(base, b_per_w)])

    return k(emb, idx)


def _epilogue_body(zn_ref, zq_ref, out_ref):
    zn = zn_ref[...]
    rows = zq_ref[...]
    n = jnp.sqrt(jnp.sum(rows * rows, axis=1, keepdims=True))
    zq = rows / jnp.maximum(n, 1e-12)
    st = zn + (zq - zn)
    out_ref[0] = st.T


def _epilogue(zn_flat, zq_flat):
    return pl.pallas_call(
        _epilogue_body,
        grid=(4,),
        in_specs=[
            pl.BlockSpec((1024, E_DIM), lambda i: (i, 0)),
            pl.BlockSpec((1024, E_DIM), lambda i: (i, 0)),
        ],
        out_specs=pl.BlockSpec((1, E_DIM, 1024), lambda i: (i, 0, 0)),
        out_shape=jax.ShapeDtypeStruct((4, E_DIM, 1024), jnp.float32),
    )(zn_flat, zq_flat)


def kernel(z, W, query_ids):
    B = z.shape[0]
    z3 = z.reshape(B, E_DIM, 1024)
    qids_flat = query_ids[:, -1].reshape(B)
    zn_flat, idx3, vq, com, ent, dcr = _vq_main(z3, W, qids_flat)
    idx = idx3.reshape(N_TOK)
    zq_flat = _sc_gather(W, idx)
    zq_out = _epilogue(zn_flat, zq_flat).reshape(B, E_DIM, 32, 32)
    return (zq_out, vq[0, 0], com[0, 0], ent[0, 0], dcr[0, 0], idx)
